# baseline (speedup 1.0000x reference)
import functools

import jax
import jax.numpy as jnp
from jax import lax
from jax.experimental import pallas as pl
from jax.experimental.pallas import tpu as pltpu
from jax.experimental.pallas import tpu_sc as plsc

_N = 10000
_E = 320000
_G = 64
_FIN = 128
_NC = 2
_NS = 16
_NW = _NC * _NS
_CH = 128
_F = 128

_NR = 5056
_ACC = 5120
_ZROWS = 64
_CPT = 312

_EPW = _E // _NW
_NCH_E = -(-_EPW // _CH)
_EPS = _E // _NS
_NCH_C = -(-_EPS // _CH)
_NCH_P = 3
_PB = 312
_HP = 10112
_DEGB = _N + 16

_BR = 1000


def _vsmesh():
    return plsc.VectorSubcoreMesh(core_axis_name="c", subcore_axis_name="s",
                                  num_cores=_NC, num_subcores=_NS)


def _fill(buf, rows, width, value):
    @pl.loop(0, rows)
    def _(i):
        @pl.loop(0, width // 16)
        def _(j):
            buf[i, pl.ds(j * 16, 16)] = jnp.full((16,), value, jnp.float32)


def _zero_slice(acc, zbuf, base, nrows):
    nfull, rem = nrows // _ZROWS, nrows % _ZROWS
    if nfull:
        @pl.loop(0, nfull)
        def _(k):
            pltpu.sync_copy(zbuf, acc.at[pl.ds(base + k * _ZROWS, _ZROWS)])
    if rem:
        pltpu.sync_copy(zbuf.at[pl.ds(0, rem)],
                        acc.at[pl.ds(base + nfull * _ZROWS, rem)])


def _copy_out(acc, zbuf, out, src_base, dst_base, nrows):
    nfull, rem = nrows // _ZROWS, nrows % _ZROWS
    if nfull:
        @pl.loop(0, nfull)
        def _(k):
            pltpu.sync_copy(acc.at[pl.ds(src_base + k * _ZROWS, _ZROWS)], zbuf)
            pltpu.sync_copy(zbuf, out.at[pl.ds(dst_base + k * _ZROWS, _ZROWS)])
    if rem:
        b = nfull * _ZROWS
        pltpu.sync_copy(acc.at[pl.ds(src_base + b, rem)],
                        zbuf.at[pl.ds(0, rem)])
        pltpu.sync_copy(zbuf.at[pl.ds(0, rem)],
                        out.at[pl.ds(dst_base + b, rem)])


@functools.cache
def _make_agg(nph):
    @functools.partial(
        pl.kernel,
        out_type=jax.ShapeDtypeStruct((nph * _HP, _F), jnp.float32),
        mesh=_vsmesh(),
        scratch_types=[
            pltpu.VMEM((_NCH_C, _CH), jnp.int32),
            pltpu.VMEM((_NCH_C, _CH), jnp.int32),
            pltpu.VMEM((_CH, _F), jnp.float32),
            pltpu.VMEM((_CH, _F), jnp.float32),
            pltpu.VMEM((_ZROWS, _F), jnp.float32),
            pltpu.VMEM((_ZROWS, _F), jnp.float32),
            pltpu.VMEM_SHARED((_ACC, _F), jnp.float32),
            pltpu.SemaphoreType.DMA,
            pltpu.SemaphoreType.DMA,
            pltpu.SemaphoreType.DMA,
            pltpu.SemaphoreType.DMA,
        ],
    )
    def agg(table, src_idx, dst_idx, out, sidx, didx, buf0, buf1,
            zbuf, cobuf, acc, g0, g1, s0, s1):
        c = lax.axis_index("c")
        s = lax.axis_index("s")
        wid = s * _NC + c
        pltpu.sync_copy(src_idx.at[wid], sidx)
        pltpu.sync_copy(dst_idx.at[wid], didx)
        _fill(zbuf, _ZROWS, _F, 0.0)

        lo = c * _NR

        @pl.loop(0, _NCH_C)
        def _(r):
            @pl.loop(0, _CH // 16)
            def _(q):
                sl = pl.ds(q * 16, 16)
                d = didx[r, sl]
                t = d - lo
                ok = (t >= 0) & (t < _NR)
                didx[r, sl] = jnp.where(ok, t, _NR + (d & 63))

        for p in range(nph):
            if p:
                @pl.loop(0, _NCH_C)
                def _(r):
                    @pl.loop(0, _CH // 16)
                    def _(q):
                        sl = pl.ds(q * 16, 16)
                        sidx[r, sl] = sidx[r, sl] + _N

            _zero_slice(acc, zbuf, s * (_ACC // _NS), _ACC // _NS)
            plsc.subcore_barrier()

            pltpu.async_copy(table.at[sidx.at[0]], buf0, g0)
            pltpu.async_copy(table.at[sidx.at[1]], buf1, g1)

            @pl.loop(0, (_NCH_C - 1) // 2)
            def _(k):
                j0 = 2 * k
                j1 = j0 + 1
                pltpu.make_async_copy(
                    table.at[sidx.at[j0]], buf0, g0).wait()
                pltpu.async_copy(buf0, acc.at[didx.at[j0]], s0, add=True)
                pltpu.make_async_copy(
                    table.at[sidx.at[j1]], buf1, g1).wait()
                pltpu.async_copy(buf1, acc.at[didx.at[j1]], s1, add=True)
                pltpu.make_async_copy(
                    buf0, acc.at[didx.at[j0]], s0).wait()
                pltpu.async_copy(table.at[sidx.at[j0 + 2]], buf0, g0)

                @pl.when(j1 + 2 < _NCH_C)
                def _():
                    pltpu.make_async_copy(
                        buf1, acc.at[didx.at[j1]], s1).wait()
                    pltpu.async_copy(table.at[sidx.at[j1 + 2]], buf1, g1)

            jl = _NCH_C - 1
            pltpu.make_async_copy(table.at[sidx.at[jl]], buf0, g0).wait()
            pltpu.sync_copy(buf0, acc.at[didx.at[jl]], add=True)
            pltpu.make_async_copy(
                buf1, acc.at[didx.at[jl - 1]], s1).wait()

            plsc.subcore_barrier()
            base = p * _HP + c * _NR
            _copy_out(acc, cobuf, out, s * _CPT, base + s * _CPT, _CPT)

            @pl.when(s == _NS - 1)
            def _():
                _copy_out(acc, cobuf, out, _NS * _CPT, base + _NS * _CPT, 64)

            if p + 1 < nph:
                plsc.subcore_barrier()

    return agg


def _make_deg():
    @functools.partial(
        pl.kernel,
        out_type=jax.ShapeDtypeStruct((_HP, _F), jnp.float32),
        mesh=_vsmesh(),
        scratch_types=[
            pltpu.VMEM((_NCH_C, _CH), jnp.int32),
            pltpu.VMEM((_CH, _F), jnp.float32),
            pltpu.VMEM((_ZROWS, _F), jnp.float32),
            pltpu.VMEM_SHARED((_ACC, _F), jnp.float32),
        ],
    )
    def deg(dst_idx, out, didx, ones, zbuf, acc):
        c = lax.axis_index("c")
        s = lax.axis_index("s")
        wid = s * _NC + c
        pltpu.sync_copy(dst_idx.at[wid], didx)
        _fill(ones, _CH, _F, 1.0)
        _fill(zbuf, _ZROWS, _F, 0.0)

        lo = c * _NR

        @pl.loop(0, _NCH_C)
        def _(r):
            @pl.loop(0, _CH // 16)
            def _(q):
                sl = pl.ds(q * 16, 16)
                d = didx[r, sl]
                t = d - lo
                ok = (t >= 0) & (t < _NR)
                didx[r, sl] = jnp.where(ok, t, _NR + (d & 63))

        _zero_slice(acc, zbuf, s * (_ACC // _NS), _ACC // _NS)
        plsc.subcore_barrier()

        @pl.loop(0, _NCH_C)
        def _(j):
            pltpu.sync_copy(ones, acc.at[didx.at[j]], add=True)

        plsc.subcore_barrier()
        _copy_out(acc, zbuf, out, s * _CPT, c * _NR + s * _CPT, _CPT)

        @pl.when(s == _NS - 1)
        def _():
            _copy_out(acc, zbuf, out, _NS * _CPT, c * _NR + _NS * _CPT, 64)

    return deg


def _make_pool():
    pacc = 128

    @functools.partial(
        pl.kernel,
        out_type=(jax.ShapeDtypeStruct((_NC * _G, _F), jnp.float32),
                  jax.ShapeDtypeStruct((_NC * _G, _F), jnp.float32)),
        mesh=_vsmesh(),
        scratch_types=[
            pltpu.VMEM((_NCH_P, _CH), jnp.int32),
            pltpu.VMEM((_CH, _F), jnp.float32),
            pltpu.VMEM((_CH, _F), jnp.float32),
            pltpu.VMEM((_ZROWS, _F), jnp.float32),
            pltpu.VMEM_SHARED((pacc, _F), jnp.float32),
            pltpu.VMEM_SHARED((pacc, _F), jnp.float32),
        ],
    )
    def pool(h, bidx_in, sums_out, cnt_out, bidx, buf, ones, zbuf, sacc, cacc):
        c = lax.axis_index("c")
        s = lax.axis_index("s")
        wid = s * _NC + c
        pltpu.sync_copy(bidx_in.at[wid], bidx)
        _fill(ones, _CH, _F, 1.0)
        _fill(zbuf, _ZROWS, _F, 0.0)
        _zero_slice(sacc, zbuf, s * (pacc // _NS), pacc // _NS)
        _zero_slice(cacc, zbuf, s * (pacc // _NS), pacc // _NS)
        plsc.subcore_barrier()

        r0 = wid * _PB

        @pl.loop(0, _NCH_P)
        def _(j):
            pltpu.sync_copy(h.at[pl.ds(r0 + j * _CH, _CH)], buf)
            pltpu.sync_copy(buf, sacc.at[bidx.at[j]], add=True)
            pltpu.sync_copy(ones, cacc.at[bidx.at[j]], add=True)

        plsc.subcore_barrier()

        @pl.when(s == 0)
        def _():
            pltpu.sync_copy(sacc.at[pl.ds(0, _G)], zbuf)
            pltpu.sync_copy(zbuf, sums_out.at[pl.ds(c * _G, _G)])
            pltpu.sync_copy(cacc.at[pl.ds(0, _G)], zbuf)
            pltpu.sync_copy(zbuf, cnt_out.at[pl.ds(c * _G, _G)])

    return pool


_deg_call = _make_deg()
_pool_call = _make_pool()


def _rowspec(width=_F):
    return pl.BlockSpec((_BR, width), lambda i: (i, 0))


def _fullspec(shape):
    nd = len(shape)
    return pl.BlockSpec(shape, lambda i: (0,) * nd)


def _pairspec():
    return pl.BlockSpec((_NC, _BR, _F), lambda i: (0, i, 0))


def _tc_mm(x, W1, Wlin, blin):
    def body(xr, w1, wl, bl, xw_o, id_o):
        xv = xr[...]
        xw_o[...] = jnp.dot(xv, w1[...], preferred_element_type=jnp.float32)
        id_o[...] = (jnp.dot(xv, wl[...], preferred_element_type=jnp.float32)
                     + bl[...])

    return pl.pallas_call(
        body,
        grid=(_N // _BR,),
        in_specs=[_rowspec(), _fullspec((_FIN, _F)), _fullspec((_FIN, _F)),
                  _fullspec((1, _F))],
        out_specs=[_rowspec(), _rowspec()],
        out_shape=[jax.ShapeDtypeStruct((_N, _F), jnp.float32)] * 2,
    )(x, W1, Wlin, blin.reshape(1, _F))


def _tc_scale(degp, xw1):
    def body(dp, xw, s_o, xs_o):
        dv = dp[...][:_N, 0:1] + 1.0
        sv = lax.rsqrt(dv)
        s_o[...] = jnp.broadcast_to(sv, (_N, 16))
        xs_o[...] = sv * xw[...]

    return pl.pallas_call(
        body,
        out_shape=[jax.ShapeDtypeStruct((_N, 16), jnp.float32),
                   jax.ShapeDtypeStruct((_N, _F), jnp.float32)],
    )(degp, xw1)


def _tc_layer(agg, xs, s, b, W, fout):
    def body(a, xsr, sr, br, wr, o):
        sc = sr[:, 0:1]
        h = jnp.maximum(sc * (a[...] + xsr[...]) + br[...], 0.0)
        xw = sc * jnp.dot(h, wr[...], preferred_element_type=jnp.float32)
        if fout == _F:
            o[...] = xw
        else:
            o[0] = xw[:, :_F]
            o[1] = xw[:, _F:]

    out_spec = _rowspec() if fout == _F else _pairspec()
    out_shape = (jax.ShapeDtypeStruct((_N, _F), jnp.float32) if fout == _F
                 else jax.ShapeDtypeStruct((_NC, _N, _F), jnp.float32))
    return pl.pallas_call(
        body,
        grid=(_N // _BR,),
        in_specs=[_rowspec(), _rowspec(), _rowspec(16),
                  _fullspec((1, _F)), _fullspec((_F, fout))],
        out_specs=out_spec,
        out_shape=out_shape,
    )(agg, xs, s, b.reshape(1, _F), W)


def _tc_layer_cs(agg, xs, s, b, W):
    def body(a, xsr, sr, br, wr, o):
        sc = sr[:, 0:1]
        pre = jnp.concatenate([a[0] + xsr[0], a[1] + xsr[1]], axis=1)
        h = jnp.maximum(sc * pre + br[...], 0.0)
        o[...] = sc * jnp.dot(h, wr[...], preferred_element_type=jnp.float32)

    return pl.pallas_call(
        body,
        grid=(_N // _BR,),
        in_specs=[_pairspec(), _pairspec(), _rowspec(16),
                  _fullspec((1, 2 * _F)), _fullspec((2 * _F, _F))],
        out_specs=_rowspec(),
        out_shape=jax.ShapeDtypeStruct((_N, _F), jnp.float32),
    )(agg, xs, s, b.reshape(1, 2 * _F), W)


def _tc_final(agg, xs, s, b, idt):
    def body(a, xsr, sr, br, idr, o):
        sc = sr[:, 0:1]
        o[...] = sc * (a[...] + xsr[...]) + br[...] + idr[...]

    return pl.pallas_call(
        body,
        grid=(_N // _BR,),
        in_specs=[_rowspec(), _rowspec(), _rowspec(16),
                  _fullspec((1, _F)), _rowspec()],
        out_specs=_rowspec(),
        out_shape=jax.ShapeDtypeStruct((_HP, _F), jnp.float32),
    )(agg, xs, s, b.reshape(1, _F), idt)


def _tc_div(sums, cnts):
    def body(sa, ca, o):
        tot = sa[0] + sa[1]
        cnt = (ca[0] + ca[1])[:, 0:1]
        o[...] = tot / jnp.maximum(cnt, 1.0)

    return pl.pallas_call(
        body,
        out_shape=jax.ShapeDtypeStruct((_G, _F), jnp.float32),
    )(sums, cnts)


def kernel(x, edge_index, batch, W1, b1, W2, b2, W3, b3, W4, b4, Wlin, blin):
    i32 = jnp.int32
    src = edge_index[0].astype(i32)
    dst = edge_index[1].astype(i32)

    pad_c = _NCH_C * _CH - _EPS
    s16 = jnp.concatenate(
        [src.reshape(_NS, _EPS), jnp.zeros((_NS, pad_c), i32)], axis=1)
    d16 = jnp.concatenate(
        [dst.reshape(_NS, _EPS), jnp.full((_NS, pad_c), _N, i32)], axis=1)
    sw2 = jnp.stack([s16, s16], axis=1).reshape(_NW, _NCH_C, _CH)
    dw2 = jnp.stack([d16, d16], axis=1).reshape(_NW, _NCH_C, _CH)

    w = jnp.arange(_NW, dtype=i32)
    base = w * _PB
    cnt = jnp.where(w == _NW - 1, _N - (_NW - 1) * _PB, _PB)
    k = jnp.arange(_NCH_P * _CH, dtype=i32)
    pos = base[:, None] + k[None, :]
    bp = jnp.where(k[None, :] < cnt[:, None],
                   batch.astype(i32)[jnp.clip(pos, 0, _N - 1)],
                   _G).reshape(_NW, _NCH_P, _CH)

    agg1 = _make_agg(1)
    agg2 = _make_agg(2)

    degp = _deg_call(dw2)
    xw1, idt = _tc_mm(x, W1, Wlin, blin)
    sN, xs1 = _tc_scale(degp, xw1)

    a1 = agg1(xs1, sw2, dw2)
    xs2 = _tc_layer(a1, xs1, sN, b1, W2, _F)
    a2 = agg1(xs2, sw2, dw2)
    xs3 = _tc_layer(a2, xs2, sN, b2, W3, 2 * _F)
    a3 = agg2(xs3.reshape(2 * _N, _F), sw2, dw2).reshape(_NC, _HP, _F)
    xs4 = _tc_layer_cs(a3, xs3, sN, b3, W4)
    a4 = agg1(xs4, sw2, dw2)
    hp = _tc_final(a4, xs4, sN, b4, idt)

    sums, cnts = _pool_call(hp, bp)
    return _tc_div(sums.reshape(_NC, _G, _F), cnts.reshape(_NC, _G, _F))


# speedup vs baseline: 1.2091x; 1.2091x over previous
import functools

import jax
import jax.numpy as jnp
from jax import lax
from jax.experimental import pallas as pl
from jax.experimental.pallas import tpu as pltpu
from jax.experimental.pallas import tpu_sc as plsc

_N = 10000
_E = 320000
_G = 64
_FIN = 128
_NC = 2
_NS = 16
_NW = _NC * _NS
_CH = 128
_F = 128

_NR = 5056
_ACC = 5120
_ZROWS = 64
_CPT = 312

_EPW = _E // _NW
_NCH_E = -(-_EPW // _CH)
_EPS = _E // _NS
_NCH_C = -(-_EPS // _CH)
_NCH_P = 3
_PB = 312
_HP = 10112
_DEGB = _N + 16

_BR = 1000


def _vsmesh():
    return plsc.VectorSubcoreMesh(core_axis_name="c", subcore_axis_name="s",
                                  num_cores=_NC, num_subcores=_NS)


def _fill(buf, rows, width, value):
    @pl.loop(0, rows)
    def _(i):
        @pl.loop(0, width // 16)
        def _(j):
            buf[i, pl.ds(j * 16, 16)] = jnp.full((16,), value, jnp.float32)


def _zero_slice(acc, zbuf, base, nrows):
    nfull, rem = nrows // _ZROWS, nrows % _ZROWS
    if nfull:
        @pl.loop(0, nfull)
        def _(k):
            pltpu.sync_copy(zbuf, acc.at[pl.ds(base + k * _ZROWS, _ZROWS)])
    if rem:
        pltpu.sync_copy(zbuf.at[pl.ds(0, rem)],
                        acc.at[pl.ds(base + nfull * _ZROWS, rem)])


def _copy_out(acc, zbuf, out, src_base, dst_base, nrows):
    nfull, rem = nrows // _ZROWS, nrows % _ZROWS
    if nfull:
        @pl.loop(0, nfull)
        def _(k):
            pltpu.sync_copy(acc.at[pl.ds(src_base + k * _ZROWS, _ZROWS)], zbuf)
            pltpu.sync_copy(zbuf, out.at[pl.ds(dst_base + k * _ZROWS, _ZROWS)])
    if rem:
        b = nfull * _ZROWS
        pltpu.sync_copy(acc.at[pl.ds(src_base + b, rem)],
                        zbuf.at[pl.ds(0, rem)])
        pltpu.sync_copy(zbuf.at[pl.ds(0, rem)],
                        out.at[pl.ds(dst_base + b, rem)])


@functools.cache
def _make_agg(nph):
    @functools.partial(
        pl.kernel,
        out_type=jax.ShapeDtypeStruct((nph * _HP, _F), jnp.float32),
        mesh=_vsmesh(),
        scratch_types=[
            pltpu.VMEM((_NCH_C, _CH), jnp.int32),
            pltpu.VMEM((_NCH_C, _CH), jnp.int32),
            pltpu.VMEM((_CH, _F), jnp.float32),
            pltpu.VMEM((_CH, _F), jnp.float32),
            pltpu.VMEM((_ZROWS, _F), jnp.float32),
            pltpu.VMEM((_ZROWS, _F), jnp.float32),
            pltpu.VMEM_SHARED((_ACC, _F), jnp.float32),
            pltpu.SemaphoreType.DMA,
            pltpu.SemaphoreType.DMA,
        ],
    )
    def agg(table, src_idx, dst_idx, out, sidx, didx, buf0, buf1,
            zbuf, cobuf, acc, sem0, sem1):
        c = lax.axis_index("c")
        s = lax.axis_index("s")
        wid = s * _NC + c
        pltpu.sync_copy(src_idx.at[wid], sidx)
        pltpu.sync_copy(dst_idx.at[wid], didx)
        _fill(zbuf, _ZROWS, _F, 0.0)

        lo = c * _NR

        @pl.loop(0, _NCH_C)
        def _(r):
            @pl.loop(0, _CH // 16)
            def _(q):
                sl = pl.ds(q * 16, 16)
                d = didx[r, sl]
                t = d - lo
                ok = (t >= 0) & (t < _NR)
                didx[r, sl] = jnp.where(ok, t, _NR + (d & 63))

        for p in range(nph):
            if p:
                @pl.loop(0, _NCH_C)
                def _(r):
                    @pl.loop(0, _CH // 16)
                    def _(q):
                        sl = pl.ds(q * 16, 16)
                        sidx[r, sl] = sidx[r, sl] + _N

            _zero_slice(acc, zbuf, s * (_ACC // _NS), _ACC // _NS)
            plsc.subcore_barrier()

            pltpu.async_copy(table.at[sidx.at[0]], buf0, sem0)

            @pl.loop(0, (_NCH_C - 1) // 2)
            def _(k):
                j0 = 2 * k
                pltpu.async_copy(table.at[sidx.at[j0 + 1]], buf1, sem1)
                pltpu.make_async_copy(
                    table.at[sidx.at[j0]], buf0, sem0).wait()
                pltpu.sync_copy(buf0, acc.at[didx.at[j0]], add=True)
                pltpu.async_copy(table.at[sidx.at[j0 + 2]], buf0, sem0)
                pltpu.make_async_copy(
                    table.at[sidx.at[j0 + 1]], buf1, sem1).wait()
                pltpu.sync_copy(buf1, acc.at[didx.at[j0 + 1]], add=True)

            pltpu.make_async_copy(
                table.at[sidx.at[_NCH_C - 1]], buf0, sem0).wait()
            pltpu.sync_copy(buf0, acc.at[didx.at[_NCH_C - 1]], add=True)

            plsc.subcore_barrier()
            base = p * _HP + c * _NR
            _copy_out(acc, cobuf, out, s * _CPT, base + s * _CPT, _CPT)

            @pl.when(s == _NS - 1)
            def _():
                _copy_out(acc, cobuf, out, _NS * _CPT, base + _NS * _CPT, 64)

            if p + 1 < nph:
                plsc.subcore_barrier()

    return agg


def _make_deg():
    @functools.partial(
        pl.kernel,
        out_type=jax.ShapeDtypeStruct((_HP, _F), jnp.float32),
        mesh=_vsmesh(),
        scratch_types=[
            pltpu.VMEM((_NCH_C, _CH), jnp.int32),
            pltpu.VMEM((_CH, _F), jnp.float32),
            pltpu.VMEM((_ZROWS, _F), jnp.float32),
            pltpu.VMEM_SHARED((_ACC, _F), jnp.float32),
        ],
    )
    def deg(dst_idx, out, didx, ones, zbuf, acc):
        c = lax.axis_index("c")
        s = lax.axis_index("s")
        wid = s * _NC + c
        pltpu.sync_copy(dst_idx.at[wid], didx)
        _fill(ones, _CH, _F, 1.0)
        _fill(zbuf, _ZROWS, _F, 0.0)

        lo = c * _NR

        @pl.loop(0, _NCH_C)
        def _(r):
            @pl.loop(0, _CH // 16)
            def _(q):
                sl = pl.ds(q * 16, 16)
                d = didx[r, sl]
                t = d - lo
                ok = (t >= 0) & (t < _NR)
                didx[r, sl] = jnp.where(ok, t, _NR + (d & 63))

        _zero_slice(acc, zbuf, s * (_ACC // _NS), _ACC // _NS)
        plsc.subcore_barrier()

        @pl.loop(0, _NCH_C)
        def _(j):
            pltpu.sync_copy(ones, acc.at[didx.at[j]], add=True)

        plsc.subcore_barrier()
        _copy_out(acc, zbuf, out, s * _CPT, c * _NR + s * _CPT, _CPT)

        @pl.when(s == _NS - 1)
        def _():
            _copy_out(acc, zbuf, out, _NS * _CPT, c * _NR + _NS * _CPT, 64)

    return deg


def _make_pool():
    pacc = 128

    @functools.partial(
        pl.kernel,
        out_type=(jax.ShapeDtypeStruct((_NC * _G, _F), jnp.float32),
                  jax.ShapeDtypeStruct((_NC * _G, _F), jnp.float32)),
        mesh=_vsmesh(),
        scratch_types=[
            pltpu.VMEM((_NCH_P, _CH), jnp.int32),
            pltpu.VMEM((_CH, _F), jnp.float32),
            pltpu.VMEM((_CH, _F), jnp.float32),
            pltpu.VMEM((_ZROWS, _F), jnp.float32),
            pltpu.VMEM_SHARED((pacc, _F), jnp.float32),
            pltpu.VMEM_SHARED((pacc, _F), jnp.float32),
        ],
    )
    def pool(h, bidx_in, sums_out, cnt_out, bidx, buf, ones, zbuf, sacc, cacc):
        c = lax.axis_index("c")
        s = lax.axis_index("s")
        wid = s * _NC + c
        pltpu.sync_copy(bidx_in.at[wid], bidx)
        _fill(ones, _CH, _F, 1.0)
        _fill(zbuf, _ZROWS, _F, 0.0)
        _zero_slice(sacc, zbuf, s * (pacc // _NS), pacc // _NS)
        _zero_slice(cacc, zbuf, s * (pacc // _NS), pacc // _NS)
        plsc.subcore_barrier()

        r0 = wid * _PB

        @pl.loop(0, _NCH_P)
        def _(j):
            pltpu.sync_copy(h.at[pl.ds(r0 + j * _CH, _CH)], buf)
            pltpu.sync_copy(buf, sacc.at[bidx.at[j]], add=True)
            pltpu.sync_copy(ones, cacc.at[bidx.at[j]], add=True)

        plsc.subcore_barrier()

        @pl.when(s == 0)
        def _():
            pltpu.sync_copy(sacc.at[pl.ds(0, _G)], zbuf)
            pltpu.sync_copy(zbuf, sums_out.at[pl.ds(c * _G, _G)])
            pltpu.sync_copy(cacc.at[pl.ds(0, _G)], zbuf)
            pltpu.sync_copy(zbuf, cnt_out.at[pl.ds(c * _G, _G)])

    return pool


_deg_call = _make_deg()
_pool_call = _make_pool()


def _rowspec(width=_F):
    return pl.BlockSpec((_BR, width), lambda i: (i, 0))


def _fullspec(shape):
    nd = len(shape)
    return pl.BlockSpec(shape, lambda i: (0,) * nd)


def _pairspec():
    return pl.BlockSpec((_NC, _BR, _F), lambda i: (0, i, 0))


def _tc_mm(x, W1, Wlin, blin):
    def body(xr, w1, wl, bl, xw_o, id_o):
        xv = xr[...]
        xw_o[...] = jnp.dot(xv, w1[...], preferred_element_type=jnp.float32)
        id_o[...] = (jnp.dot(xv, wl[...], preferred_element_type=jnp.float32)
                     + bl[...])

    return pl.pallas_call(
        body,
        grid=(_N // _BR,),
        in_specs=[_rowspec(), _fullspec((_FIN, _F)), _fullspec((_FIN, _F)),
                  _fullspec((1, _F))],
        out_specs=[_rowspec(), _rowspec()],
        out_shape=[jax.ShapeDtypeStruct((_N, _F), jnp.float32)] * 2,
    )(x, W1, Wlin, blin.reshape(1, _F))


def _tc_scale(degp, xw1):
    def body(dp, xw, s_o, xs_o):
        dv = dp[...][:_N, 0:1] + 1.0
        sv = lax.rsqrt(dv)
        s_o[...] = jnp.broadcast_to(sv, (_N, 16))
        xs_o[...] = sv * xw[...]

    return pl.pallas_call(
        body,
        out_shape=[jax.ShapeDtypeStruct((_N, 16), jnp.float32),
                   jax.ShapeDtypeStruct((_N, _F), jnp.float32)],
    )(degp, xw1)


def _tc_layer(agg, xs, s, b, W, fout):
    def body(a, xsr, sr, br, wr, o):
        sc = sr[:, 0:1]
        h = jnp.maximum(sc * (a[...] + xsr[...]) + br[...], 0.0)
        xw = sc * jnp.dot(h, wr[...], preferred_element_type=jnp.float32)
        if fout == _F:
            o[...] = xw
        else:
            o[0] = xw[:, :_F]
            o[1] = xw[:, _F:]

    out_spec = _rowspec() if fout == _F else _pairspec()
    out_shape = (jax.ShapeDtypeStruct((_N, _F), jnp.float32) if fout == _F
                 else jax.ShapeDtypeStruct((_NC, _N, _F), jnp.float32))
    return pl.pallas_call(
        body,
        grid=(_N // _BR,),
        in_specs=[_rowspec(), _rowspec(), _rowspec(16),
                  _fullspec((1, _F)), _fullspec((_F, fout))],
        out_specs=out_spec,
        out_shape=out_shape,
    )(agg, xs, s, b.reshape(1, _F), W)


def _tc_layer_cs(agg, xs, s, b, W):
    def body(a, xsr, sr, br, wr, o):
        sc = sr[:, 0:1]
        pre = jnp.concatenate([a[0] + xsr[0], a[1] + xsr[1]], axis=1)
        h = jnp.maximum(sc * pre + br[...], 0.0)
        o[...] = sc * jnp.dot(h, wr[...], preferred_element_type=jnp.float32)

    return pl.pallas_call(
        body,
        grid=(_N // _BR,),
        in_specs=[_pairspec(), _pairspec(), _rowspec(16),
                  _fullspec((1, 2 * _F)), _fullspec((2 * _F, _F))],
        out_specs=_rowspec(),
        out_shape=jax.ShapeDtypeStruct((_N, _F), jnp.float32),
    )(agg, xs, s, b.reshape(1, 2 * _F), W)


def _tc_final(agg, xs, s, b, idt):
    def body(a, xsr, sr, br, idr, o):
        sc = sr[:, 0:1]
        o[...] = sc * (a[...] + xsr[...]) + br[...] + idr[...]

    return pl.pallas_call(
        body,
        grid=(_N // _BR,),
        in_specs=[_rowspec(), _rowspec(), _rowspec(16),
                  _fullspec((1, _F)), _rowspec()],
        out_specs=_rowspec(),
        out_shape=jax.ShapeDtypeStruct((_HP, _F), jnp.float32),
    )(agg, xs, s, b.reshape(1, _F), idt)


def _tc_div(sums, cnts):
    def body(sa, ca, o):
        tot = sa[0] + sa[1]
        cnt = (ca[0] + ca[1])[:, 0:1]
        o[...] = tot / jnp.maximum(cnt, 1.0)

    return pl.pallas_call(
        body,
        out_shape=jax.ShapeDtypeStruct((_G, _F), jnp.float32),
    )(sums, cnts)


def kernel(x, edge_index, batch, W1, b1, W2, b2, W3, b3, W4, b4, Wlin, blin):
    i32 = jnp.int32
    src = edge_index[0].astype(i32)
    dst = edge_index[1].astype(i32)

    pad_c = _NCH_C * _CH - _EPS
    s16 = jnp.concatenate(
        [src.reshape(_NS, _EPS), jnp.zeros((_NS, pad_c), i32)], axis=1)
    d16 = jnp.concatenate(
        [dst.reshape(_NS, _EPS), jnp.full((_NS, pad_c), _N, i32)], axis=1)
    sw2 = jnp.stack([s16, s16], axis=1).reshape(_NW, _NCH_C, _CH)
    dw2 = jnp.stack([d16, d16], axis=1).reshape(_NW, _NCH_C, _CH)

    w = jnp.arange(_NW, dtype=i32)
    base = w * _PB
    cnt = jnp.where(w == _NW - 1, _N - (_NW - 1) * _PB, _PB)
    k = jnp.arange(_NCH_P * _CH, dtype=i32)
    pos = base[:, None] + k[None, :]
    bp = jnp.where(k[None, :] < cnt[:, None],
                   batch.astype(i32)[jnp.clip(pos, 0, _N - 1)],
                   _G).reshape(_NW, _NCH_P, _CH)

    agg1 = _make_agg(1)
    agg2 = _make_agg(2)

    degp = _deg_call(dw2)
    xw1, idt = _tc_mm(x, W1, Wlin, blin)
    sN, xs1 = _tc_scale(degp, xw1)

    a1 = agg1(xs1, sw2, dw2)
    xs2 = _tc_layer(a1, xs1, sN, b1, W2, _F)
    a2 = agg1(xs2, sw2, dw2)
    xs3 = _tc_layer(a2, xs2, sN, b2, W3, 2 * _F)
    a3 = agg2(xs3.reshape(2 * _N, _F), sw2, dw2).reshape(_NC, _HP, _F)
    xs4 = _tc_layer_cs(a3, xs3, sN, b3, W4)
    a4 = agg1(xs4, sw2, dw2)
    hp = _tc_final(a4, xs4, sN, b4, idt)

    sums, cnts = _pool_call(hp, bp)
    return _tc_div(sums.reshape(_NC, _G, _F), cnts.reshape(_NC, _G, _F))


# speedup vs baseline: 1.2094x; 1.0002x over previous
import functools

import jax
import jax.numpy as jnp
from jax import lax
from jax.experimental import pallas as pl
from jax.experimental.pallas import tpu as pltpu
from jax.experimental.pallas import tpu_sc as plsc

_N = 10000
_E = 320000
_G = 64
_FIN = 128
_NC = 2
_NS = 16
_NW = _NC * _NS
_CH = 128
_F = 128

_NR = 5056
_ACC = 5120
_ZROWS = 64
_CPT = 312

_EPW = _E // _NW
_NCH_E = -(-_EPW // _CH)
_EPS = _E // _NS
_NCH_C = -(-_EPS // _CH)
_NCH_P = 3
_PB = 312
_HP = 10112

_BR = 1000


def _vsmesh():
    return plsc.VectorSubcoreMesh(core_axis_name="c", subcore_axis_name="s",
                                  num_cores=_NC, num_subcores=_NS)


def _fill(buf, rows, width, value):
    @pl.loop(0, rows)
    def _(i):
        @pl.loop(0, width // 16)
        def _(j):
            buf[i, pl.ds(j * 16, 16)] = jnp.full((16,), value, jnp.float32)


def _zero_slice(acc, zbuf, base, nrows):
    nfull, rem = nrows // _ZROWS, nrows % _ZROWS
    if nfull:
        @pl.loop(0, nfull)
        def _(k):
            pltpu.sync_copy(zbuf, acc.at[pl.ds(base + k * _ZROWS, _ZROWS)])
    if rem:
        pltpu.sync_copy(zbuf.at[pl.ds(0, rem)],
                        acc.at[pl.ds(base + nfull * _ZROWS, rem)])


def _copy_out(acc, zbuf, out, src_base, dst_base, nrows):
    nfull, rem = nrows // _ZROWS, nrows % _ZROWS
    if nfull:
        @pl.loop(0, nfull)
        def _(k):
            pltpu.sync_copy(acc.at[pl.ds(src_base + k * _ZROWS, _ZROWS)], zbuf)
            pltpu.sync_copy(zbuf, out.at[pl.ds(dst_base + k * _ZROWS, _ZROWS)])
    if rem:
        b = nfull * _ZROWS
        pltpu.sync_copy(acc.at[pl.ds(src_base + b, rem)],
                        zbuf.at[pl.ds(0, rem)])
        pltpu.sync_copy(zbuf.at[pl.ds(0, rem)],
                        out.at[pl.ds(dst_base + b, rem)])


@functools.cache
def _make_agg(nph):
    @functools.partial(
        pl.kernel,
        out_type=jax.ShapeDtypeStruct((nph * _HP, _F), jnp.float32),
        mesh=_vsmesh(),
        scratch_types=[
            pltpu.VMEM((_NCH_C, _CH), jnp.int32),
            pltpu.VMEM((_NCH_C, _CH), jnp.int32),
            pltpu.VMEM((_CH, _F), jnp.float32),
            pltpu.VMEM((_CH, _F), jnp.float32),
            pltpu.VMEM((_ZROWS, _F), jnp.float32),
            pltpu.VMEM((_ZROWS, _F), jnp.float32),
            pltpu.VMEM_SHARED((_ACC, _F), jnp.float32),
            pltpu.SemaphoreType.DMA,
            pltpu.SemaphoreType.DMA,
        ],
    )
    def agg(table, src_idx, dst_idx, out, sidx, didx, buf0, buf1,
            zbuf, cobuf, acc, sem0, sem1):
        c = lax.axis_index("c")
        s = lax.axis_index("s")
        wid = s * _NC + c
        pltpu.sync_copy(src_idx.at[wid], sidx)
        pltpu.sync_copy(dst_idx.at[wid], didx)
        _fill(zbuf, _ZROWS, _F, 0.0)

        lo = c * _NR

        @pl.loop(0, _NCH_C)
        def _(r):
            @pl.loop(0, _CH // 16)
            def _(q):
                sl = pl.ds(q * 16, 16)
                d = didx[r, sl]
                t = d - lo
                ok = (t >= 0) & (t < _NR)
                didx[r, sl] = jnp.where(ok, t, _NR + (d & 63))

        for p in range(nph):
            if p:
                @pl.loop(0, _NCH_C)
                def _(r):
                    @pl.loop(0, _CH // 16)
                    def _(q):
                        sl = pl.ds(q * 16, 16)
                        sidx[r, sl] = sidx[r, sl] + _N

            _zero_slice(acc, zbuf, s * (_ACC // _NS), _ACC // _NS)
            plsc.subcore_barrier()

            pltpu.async_copy(table.at[sidx.at[0]], buf0, sem0)

            @pl.loop(0, (_NCH_C - 1) // 2)
            def _(k):
                j0 = 2 * k
                pltpu.async_copy(table.at[sidx.at[j0 + 1]], buf1, sem1)
                pltpu.make_async_copy(
                    table.at[sidx.at[j0]], buf0, sem0).wait()
                pltpu.sync_copy(buf0, acc.at[didx.at[j0]], add=True)
                pltpu.async_copy(table.at[sidx.at[j0 + 2]], buf0, sem0)
                pltpu.make_async_copy(
                    table.at[sidx.at[j0 + 1]], buf1, sem1).wait()
                pltpu.sync_copy(buf1, acc.at[didx.at[j0 + 1]], add=True)

            pltpu.make_async_copy(
                table.at[sidx.at[_NCH_C - 1]], buf0, sem0).wait()
            pltpu.sync_copy(buf0, acc.at[didx.at[_NCH_C - 1]], add=True)

            plsc.subcore_barrier()
            base = p * _HP + c * _NR
            _copy_out(acc, cobuf, out, s * _CPT, base + s * _CPT, _CPT)

            @pl.when(s == _NS - 1)
            def _():
                _copy_out(acc, cobuf, out, _NS * _CPT, base + _NS * _CPT, 64)

            if p + 1 < nph:
                plsc.subcore_barrier()

    return agg


def _make_deg():
    @functools.partial(
        pl.kernel,
        out_type=jax.ShapeDtypeStruct((_HP, _F), jnp.float32),
        mesh=_vsmesh(),
        scratch_types=[
            pltpu.VMEM((_NCH_C, _CH), jnp.int32),
            pltpu.VMEM((_CH, _F), jnp.float32),
            pltpu.VMEM((_ZROWS, _F), jnp.float32),
            pltpu.VMEM_SHARED((_ACC, _F), jnp.float32),
        ],
    )
    def deg(dst_idx, out, didx, ones, zbuf, acc):
        c = lax.axis_index("c")
        s = lax.axis_index("s")
        wid = s * _NC + c
        pltpu.sync_copy(dst_idx.at[wid], didx)
        _fill(ones, _CH, _F, 1.0)
        _fill(zbuf, _ZROWS, _F, 0.0)

        lo = c * _NR

        @pl.loop(0, _NCH_C)
        def _(r):
            @pl.loop(0, _CH // 16)
            def _(q):
                sl = pl.ds(q * 16, 16)
                d = didx[r, sl]
                t = d - lo
                ok = (t >= 0) & (t < _NR)
                didx[r, sl] = jnp.where(ok, t, _NR + (d & 63))

        _zero_slice(acc, zbuf, s * (_ACC // _NS), _ACC // _NS)
        plsc.subcore_barrier()

        @pl.loop(0, _NCH_C)
        def _(j):
            pltpu.sync_copy(ones, acc.at[didx.at[j]], add=True)

        plsc.subcore_barrier()
        _copy_out(acc, zbuf, out, s * _CPT, c * _NR + s * _CPT, _CPT)

        @pl.when(s == _NS - 1)
        def _():
            _copy_out(acc, zbuf, out, _NS * _CPT, c * _NR + _NS * _CPT, 64)

    return deg


def _make_pool():
    pacc = 128

    @functools.partial(
        pl.kernel,
        out_type=(jax.ShapeDtypeStruct((_NC * _G, _F), jnp.float32),
                  jax.ShapeDtypeStruct((_NC * _G, _F), jnp.float32)),
        mesh=_vsmesh(),
        scratch_types=[
            pltpu.VMEM((_NCH_P, _CH), jnp.int32),
            pltpu.VMEM((_CH, _F), jnp.float32),
            pltpu.VMEM((_CH, _F), jnp.float32),
            pltpu.VMEM((_ZROWS, _F), jnp.float32),
            pltpu.VMEM_SHARED((pacc, _F), jnp.float32),
            pltpu.VMEM_SHARED((pacc, _F), jnp.float32),
        ],
    )
    def pool(h, bidx_in, sums_out, cnt_out, bidx, buf, ones, zbuf, sacc, cacc):
        c = lax.axis_index("c")
        s = lax.axis_index("s")
        wid = s * _NC + c
        pltpu.sync_copy(bidx_in.at[wid], bidx)
        _fill(ones, _CH, _F, 1.0)
        _fill(zbuf, _ZROWS, _F, 0.0)
        _zero_slice(sacc, zbuf, s * (pacc // _NS), pacc // _NS)
        _zero_slice(cacc, zbuf, s * (pacc // _NS), pacc // _NS)
        plsc.subcore_barrier()

        r0 = wid * _PB

        @pl.loop(0, _NCH_P)
        def _(j):
            pltpu.sync_copy(h.at[pl.ds(r0 + j * _CH, _CH)], buf)
            pltpu.sync_copy(buf, sacc.at[bidx.at[j]], add=True)
            pltpu.sync_copy(ones, cacc.at[bidx.at[j]], add=True)

        plsc.subcore_barrier()

        @pl.when(s == 0)
        def _():
            pltpu.sync_copy(sacc.at[pl.ds(0, _G)], zbuf)
            pltpu.sync_copy(zbuf, sums_out.at[pl.ds(c * _G, _G)])
            pltpu.sync_copy(cacc.at[pl.ds(0, _G)], zbuf)
            pltpu.sync_copy(zbuf, cnt_out.at[pl.ds(c * _G, _G)])

    return pool


_deg_call = _make_deg()
_pool_call = _make_pool()


def _rowspec(width=_F):
    return pl.BlockSpec((_BR, width), lambda i: (i, 0))


def _fullspec(shape):
    nd = len(shape)
    return pl.BlockSpec(shape, lambda i: (0,) * nd)


def _pairspec():
    return pl.BlockSpec((_NC, _BR, _F), lambda i: (0, i, 0))


def _tc_mm(x, W1, Wlin, blin):
    def body(xr, w1, wl, bl, xw_o, id_o):
        xv = xr[...]
        xw_o[...] = jnp.dot(xv, w1[...], preferred_element_type=jnp.float32)
        id_o[...] = (jnp.dot(xv, wl[...], preferred_element_type=jnp.float32)
                     + bl[...])

    return pl.pallas_call(
        body,
        grid=(_N // _BR,),
        in_specs=[_rowspec(), _fullspec((_FIN, _F)), _fullspec((_FIN, _F)),
                  _fullspec((1, _F))],
        out_specs=[_rowspec(), _rowspec()],
        out_shape=[jax.ShapeDtypeStruct((_N, _F), jnp.float32)] * 2,
    )(x, W1, Wlin, blin.reshape(1, _F))


def _tc_scale(degp, xw1):
    def body(dp, xw, s_o, xs_o):
        dv = dp[...][:_N, 0:1] + 1.0
        sv = lax.rsqrt(dv)
        s_o[...] = jnp.broadcast_to(sv, (_N, 16))
        xs_o[...] = sv * xw[...]

    return pl.pallas_call(
        body,
        out_shape=[jax.ShapeDtypeStruct((_N, 16), jnp.float32),
                   jax.ShapeDtypeStruct((_N, _F), jnp.float32)],
    )(degp, xw1)


def _tc_layer(agg, xs, s, b, W, fout):
    def body(a, xsr, sr, br, wr, o):
        sc = sr[:, 0:1]
        h = jnp.maximum(sc * (a[...] + xsr[...]) + br[...], 0.0)
        xw = sc * jnp.dot(h, wr[...], preferred_element_type=jnp.float32)
        if fout == _F:
            o[...] = xw
        else:
            o[0] = xw[:, :_F]
            o[1] = xw[:, _F:]

    out_spec = _rowspec() if fout == _F else _pairspec()
    out_shape = (jax.ShapeDtypeStruct((_N, _F), jnp.float32) if fout == _F
                 else jax.ShapeDtypeStruct((_NC, _N, _F), jnp.float32))
    return pl.pallas_call(
        body,
        grid=(_N // _BR,),
        in_specs=[_rowspec(), _rowspec(), _rowspec(16),
                  _fullspec((1, _F)), _fullspec((_F, fout))],
        out_specs=out_spec,
        out_shape=out_shape,
    )(agg, xs, s, b.reshape(1, _F), W)


def _tc_layer_cs(agg, xs, s, b, W):
    def body(a, xsr, sr, br, wr, o):
        sc = sr[:, 0:1]
        pre = jnp.concatenate([a[0] + xsr[0], a[1] + xsr[1]], axis=1)
        h = jnp.maximum(sc * pre + br[...], 0.0)
        o[...] = sc * jnp.dot(h, wr[...], preferred_element_type=jnp.float32)

    return pl.pallas_call(
        body,
        grid=(_N // _BR,),
        in_specs=[_pairspec(), _pairspec(), _rowspec(16),
                  _fullspec((1, 2 * _F)), _fullspec((2 * _F, _F))],
        out_specs=_rowspec(),
        out_shape=jax.ShapeDtypeStruct((_N, _F), jnp.float32),
    )(agg, xs, s, b.reshape(1, 2 * _F), W)


def _tc_final(agg, xs, s, b, idt):
    def body(a, xsr, sr, br, idr, o):
        sc = sr[:, 0:1]
        o[...] = sc * (a[...] + xsr[...]) + br[...] + idr[...]

    return pl.pallas_call(
        body,
        grid=(_N // _BR,),
        in_specs=[_rowspec(), _rowspec(), _rowspec(16),
                  _fullspec((1, _F)), _rowspec()],
        out_specs=_rowspec(),
        out_shape=jax.ShapeDtypeStruct((_HP, _F), jnp.float32),
    )(agg, xs, s, b.reshape(1, _F), idt)


def _tc_div(sums, cnts):
    def body(sa, ca, o):
        tot = sa[0] + sa[1]
        cnt = (ca[0] + ca[1])[:, 0:1]
        o[...] = tot / jnp.maximum(cnt, 1.0)

    return pl.pallas_call(
        body,
        out_shape=jax.ShapeDtypeStruct((_G, _F), jnp.float32),
    )(sums, cnts)


def kernel(x, edge_index, batch, W1, b1, W2, b2, W3, b3, W4, b4, Wlin, blin):
    i32 = jnp.int32
    src = edge_index[0].astype(i32)
    dst = edge_index[1].astype(i32)

    pad_c = _NCH_C * _CH - _EPS
    s16 = jnp.concatenate(
        [src.reshape(_NS, _EPS), jnp.zeros((_NS, pad_c), i32)], axis=1)
    d16 = jnp.concatenate(
        [dst.reshape(_NS, _EPS), jnp.full((_NS, pad_c), _N, i32)], axis=1)
    sw2 = jnp.stack([s16, s16], axis=1).reshape(_NW, _NCH_C, _CH)
    dw2 = jnp.stack([d16, d16], axis=1).reshape(_NW, _NCH_C, _CH)

    w = jnp.arange(_NW, dtype=i32)
    base = w * _PB
    cnt = jnp.where(w == _NW - 1, _N - (_NW - 1) * _PB, _PB)
    k = jnp.arange(_NCH_P * _CH, dtype=i32)
    pos = base[:, None] + k[None, :]
    bp = jnp.where(k[None, :] < cnt[:, None],
                   batch.astype(i32)[jnp.clip(pos, 0, _N - 1)],
                   _G).reshape(_NW, _NCH_P, _CH)

    agg1 = _make_agg(1)
    agg2 = _make_agg(2)

    degp = _deg_call(dw2)
    xw1, idt = _tc_mm(x, W1, Wlin, blin)
    sN, xs1 = _tc_scale(degp, xw1)

    a1 = agg1(xs1, sw2, dw2)
    xs2 = _tc_layer(a1, xs1, sN, b1, W2, _F)
    a2 = agg1(xs2, sw2, dw2)
    xs3 = _tc_layer(a2, xs2, sN, b2, W3, 2 * _F)
    a3 = agg2(xs3.reshape(2 * _N, _F), sw2, dw2).reshape(_NC, _HP, _F)
    xs4 = _tc_layer_cs(a3, xs3, sN, b3, W4)
    a4 = agg1(xs4, sw2, dw2)
    hp = _tc_final(a4, xs4, sN, b4, idt)

    sums, cnts = _pool_call(hp, bp)
    return _tc_div(sums.reshape(_NC, _G, _F), cnts.reshape(_NC, _G, _F))
